# Initial kernel scaffold; baseline (speedup 1.0000x reference)
import jax
import jax.numpy as jnp
from jax.experimental import pallas as pl


def kernel(src_padded_nodes_neighbor_ids, dst_padded_nodes_neighbor_ids, src_node_ids, dst_node_ids, node_interact_times, src_padded_nodes_neighbor_times, dst_padded_nodes_neighbor_times, W1, b1, W2, b2):
    raise NotImplementedError("write your pallas kernel here")


# speedup vs baseline: 29.9511x; 29.9511x over previous
import functools

import jax
import jax.numpy as jnp
from jax import lax
from jax.experimental import pallas as pl
from jax.experimental.pallas import tpu as pltpu

EPS = 1e-06
BIG = 1e9
L = 50


def _feat_body(idsA_ref, idsB_ref, tA_ref, tB_ref, othA_ref, othB_ref,
               curt_ref, fA_ref, fB_ref, srA_ref, srB_ref):
    Bb = idsA_ref.shape[1]
    idsA = idsA_ref[...]
    idsB = idsB_ref[...]
    tA = tA_ref[...]
    tB = tB_ref[...]
    iota_i = lax.broadcasted_iota(jnp.int32, (L, Bb), 0)

    zi = jnp.zeros((L, Bb), jnp.int32)
    zf = jnp.zeros((L, Bb), jnp.float32)
    big = jnp.full((L, Bb), BIG, jnp.float32)

    def pass1(j, carry):
        (cAA, srA, mnAA, mxAA, cAB, mnAB, mxAB, lastAB,
         cBB, srB, mnBB, mxBB, cBA, mnBA, mxBA, lastBA) = carry
        aj = idsA_ref[pl.ds(j, 1), :]
        bj = idsB_ref[pl.ds(j, 1), :]
        taj = tA_ref[pl.ds(j, 1), :]
        tbj = tB_ref[pl.ds(j, 1), :]
        mAA = idsA == aj
        mAB = idsA == bj
        mBB = idsB == bj
        mBA = idsB == aj
        one = jnp.int32(1)
        cAA = jnp.where(mAA, cAA + one, cAA)
        mnAA = jnp.where(mAA, jnp.minimum(mnAA, taj), mnAA)
        mxAA = jnp.where(mAA, jnp.maximum(mxAA, taj), mxAA)
        srA = jnp.where(mAA & ((taj < tA) | ((taj == tA) & (j < iota_i))),
                        srA + one, srA)
        cAB = jnp.where(mAB, cAB + one, cAB)
        mnAB = jnp.where(mAB, jnp.minimum(mnAB, tbj), mnAB)
        mxAB = jnp.where(mAB, jnp.maximum(mxAB, tbj), mxAB)
        lastAB = jnp.where(mAB & (bj != 0), tbj, lastAB)
        cBB = jnp.where(mBB, cBB + one, cBB)
        mnBB = jnp.where(mBB, jnp.minimum(mnBB, tbj), mnBB)
        mxBB = jnp.where(mBB, jnp.maximum(mxBB, tbj), mxBB)
        srB = jnp.where(mBB & ((tbj < tB) | ((tbj == tB) & (j < iota_i))),
                        srB + one, srB)
        cBA = jnp.where(mBA, cBA + one, cBA)
        mnBA = jnp.where(mBA, jnp.minimum(mnBA, taj), mnBA)
        mxBA = jnp.where(mBA, jnp.maximum(mxBA, taj), mxBA)
        lastBA = jnp.where(mBA & (aj != 0), taj, lastBA)
        return (cAA, srA, mnAA, mxAA, cAB, mnAB, mxAB, lastAB,
                cBB, srB, mnBB, mxBB, cBA, mnBA, mxBA, lastBA)

    init = (zi, zi, big, -big, zi, big, -big, zf,
            zi, zi, big, -big, zi, big, -big, zf)
    (cAA, srA, mnAA, mxAA, cAB, mnAB, mxAB, lastAB,
     cBB, srB, mnBB, mxBB, cBA, mnBA, mxBA, lastBA) = lax.fori_loop(
        0, L, pass1, init)

    srA_ref[...] = srA
    srB_ref[...] = srB
    spAA = cAA // 2
    spAB = cAB // 2
    spBB = cBB // 2
    spBA = cBA // 2

    def pass2(j, carry):
        vspAA, vspAB, vspBB, vspBA = carry
        aj = idsA_ref[pl.ds(j, 1), :]
        bj = idsB_ref[pl.ds(j, 1), :]
        taj = tA_ref[pl.ds(j, 1), :]
        tbj = tB_ref[pl.ds(j, 1), :]
        srAj = srA_ref[pl.ds(j, 1), :]
        srBj = srB_ref[pl.ds(j, 1), :]
        vspAA = jnp.where((idsA == aj) & (srAj == spAA), taj, vspAA)
        vspAB = jnp.where((idsA == bj) & (srBj == spAB), tbj, vspAB)
        vspBB = jnp.where((idsB == bj) & (srBj == spBB), tbj, vspBB)
        vspBA = jnp.where((idsB == aj) & (srAj == spBA), taj, vspBA)
        return vspAA, vspAB, vspBB, vspBA

    vspAA, vspAB, vspBB, vspBA = lax.fori_loop(
        0, L, pass2, (zf, zf, zf, zf))

    curt = curt_ref[...]

    def side(ids, t, oth, c_s, mn_s, mx_s, vsp_s, c_o, mn_o, mx_o, vsp_o,
             last_o, f_ref):
        fc_s = c_s.astype(jnp.float32)
        fc_o = c_o.astype(jnp.float32)
        keymask = ids != 0
        n_self = fc_s
        n_other = fc_o
        is_other = (ids == oth).astype(jnp.float32)
        connects = (c_o > 0).astype(jnp.float32)
        freq_asym = jnp.where(c_o > 0, fc_s / (fc_o + EPS), 0.0)
        rec_self = curt - t
        rec_other = curt - last_o
        temp_asym = jnp.where(rec_self > EPS, rec_other / (rec_self + EPS),
                              0.0)
        iat_self = jnp.where((c_s > 1) & keymask,
                             (mx_s - mn_s) / jnp.maximum(fc_s - 1.0, 1.0),
                             0.0)
        iat_other = jnp.where((c_o > 1) & keymask,
                              (mx_o - mn_o) / jnp.maximum(fc_o - 1.0, 1.0),
                              0.0)
        iat_asym = jnp.where(iat_other > EPS, iat_self / (iat_other + EPS),
                             0.0)
        fsp_s = (c_s // 2).astype(jnp.float32)
        fsp_o = (c_o // 2).astype(jnp.float32)
        r_self = jnp.where(
            (c_s >= 4) & keymask,
            (mx_s - vsp_s) / jnp.maximum(fc_s - fsp_s - 1.0, 1.0), 0.0)
        r_other = jnp.where(
            (c_o >= 4) & keymask,
            (mx_o - vsp_o) / jnp.maximum(fc_o - fsp_o - 1.0, 1.0), 0.0)
        r_asym = jnp.where(r_other > EPS, r_self / (r_other + EPS), 0.0)
        f_ref[0, :, :] = n_self
        f_ref[1, :, :] = n_other
        f_ref[2, :, :] = is_other
        f_ref[3, :, :] = connects
        f_ref[4, :, :] = freq_asym
        f_ref[5, :, :] = temp_asym
        f_ref[6, :, :] = iat_asym
        f_ref[7, :, :] = r_asym

    side(idsA, tA, othA_ref[...], cAA, mnAA, mxAA, vspAA,
         cAB, mnAB, mxAB, vspAB, lastAB, fA_ref)
    side(idsB, tB, othB_ref[...], cBB, mnBB, mxBB, vspBB,
         cBA, mnBA, mxBA, vspBA, lastBA, fB_ref)


def _encode_body(fA_ref, fB_ref, w1_ref, b1_ref, w2_ref, b2_ref,
                 outA_ref, outB_ref):
    Bb = fA_ref.shape[2]
    w1 = w1_ref[...]
    b1 = b1_ref[...]
    w2 = w2_ref[...]
    b2 = b2_ref[...]

    def one_side(f_ref, out_ref):
        for l2 in range(L // 2):
            cols = []
            for l in (2 * l2, 2 * l2 + 1):
                g = jnp.zeros((64, Bb), jnp.float32)
                for fi in range(8):
                    row = f_ref[fi, pl.ds(l, 1), :]
                    g = g + jnp.maximum(w1 * row + b1, 0.0)
                cols.append(g)
            g2 = jnp.concatenate(cols, axis=1)
            o = lax.dot_general(g2, w2, (((0,), (0,)), ((), ())),
                                preferred_element_type=jnp.float32)
            o = o + 8.0 * b2
            out_ref[:, pl.ds(2 * l2 * 64, 128)] = jnp.concatenate(
                [o[:Bb, :], o[Bb:, :]], axis=1)

    one_side(fA_ref, outA_ref)
    one_side(fB_ref, outB_ref)


@jax.jit
def _run(idsA_t, idsB_t, tA_t, tB_t, othA, othB, curt, w1c, b1c, w2, b2r):
    B = idsA_t.shape[1]
    FBB = 256
    feat_shape = jax.ShapeDtypeStruct((8, L, B), jnp.float32)
    fA, fB = pl.pallas_call(
        _feat_body,
        grid=(B // FBB,),
        in_specs=[
            pl.BlockSpec((L, FBB), lambda i: (0, i)),
            pl.BlockSpec((L, FBB), lambda i: (0, i)),
            pl.BlockSpec((L, FBB), lambda i: (0, i)),
            pl.BlockSpec((L, FBB), lambda i: (0, i)),
            pl.BlockSpec((1, FBB), lambda i: (0, i)),
            pl.BlockSpec((1, FBB), lambda i: (0, i)),
            pl.BlockSpec((1, FBB), lambda i: (0, i)),
        ],
        out_specs=[
            pl.BlockSpec((8, L, FBB), lambda i: (0, 0, i)),
            pl.BlockSpec((8, L, FBB), lambda i: (0, 0, i)),
        ],
        out_shape=[feat_shape, feat_shape],
        scratch_shapes=[
            pltpu.VMEM((L, FBB), jnp.int32),
            pltpu.VMEM((L, FBB), jnp.int32),
        ],
    )(idsA_t, idsB_t, tA_t, tB_t, othA, othB, curt)

    EBB = 128
    out_shape = jax.ShapeDtypeStruct((B, L * 64), jnp.float32)
    outA, outB = pl.pallas_call(
        _encode_body,
        grid=(B // EBB,),
        in_specs=[
            pl.BlockSpec((8, L, EBB), lambda i: (0, 0, i)),
            pl.BlockSpec((8, L, EBB), lambda i: (0, 0, i)),
            pl.BlockSpec((64, 1), lambda i: (0, 0)),
            pl.BlockSpec((64, 1), lambda i: (0, 0)),
            pl.BlockSpec((64, 64), lambda i: (0, 0)),
            pl.BlockSpec((1, 64), lambda i: (0, 0)),
        ],
        out_specs=[
            pl.BlockSpec((EBB, L * 64), lambda i: (i, 0)),
            pl.BlockSpec((EBB, L * 64), lambda i: (i, 0)),
        ],
        out_shape=[out_shape, out_shape],
    )(fA, fB, w1c, b1c, w2, b2r)
    return outA.reshape(B, L, 64), outB.reshape(B, L, 64)


def kernel(src_padded_nodes_neighbor_ids, dst_padded_nodes_neighbor_ids,
           src_node_ids, dst_node_ids, node_interact_times,
           src_padded_nodes_neighbor_times, dst_padded_nodes_neighbor_times,
           W1, b1, W2, b2):
    idsA_t = src_padded_nodes_neighbor_ids.astype(jnp.int32).T
    idsB_t = dst_padded_nodes_neighbor_ids.astype(jnp.int32).T
    tA_t = src_padded_nodes_neighbor_times.T
    tB_t = dst_padded_nodes_neighbor_times.T
    othA = dst_node_ids.astype(jnp.int32).reshape(1, -1)
    othB = src_node_ids.astype(jnp.int32).reshape(1, -1)
    curt = node_interact_times.reshape(1, -1)
    w1c = W1.reshape(1, 64).T
    b1c = b1.reshape(64, 1)
    b2r = b2.reshape(1, 64)
    return _run(idsA_t, idsB_t, tA_t, tB_t, othA, othB, curt,
                w1c, b1c, W2, b2r)
